# Initial kernel scaffold; baseline (speedup 1.0000x reference)
import jax
import jax.numpy as jnp
from jax.experimental import pallas as pl


def kernel(edges, h, W1, b1, W2, b2):
    raise NotImplementedError("write your pallas kernel here")


# speedup vs baseline: 3.5200x; 3.5200x over previous
import functools

import jax
import jax.numpy as jnp
from jax import lax
from jax.experimental import pallas as pl
from jax.experimental.pallas import tpu as pltpu
from jax.experimental.pallas import tpu_sc as plsc

N_NODES = 10000
N_EDGES = 320000
H = 128
L = 16
KV = H // L
NW = 32
EPW = N_EDGES // NW
CH = 80
NCHUNK = EPW // CH


def _tables_body(h_ref, wa_ref, wb_ref, b1_ref, a_ref, b_ref):
    x = h_ref[...]
    a_ref[...] = (
        jnp.dot(x, wa_ref[...], preferred_element_type=jnp.float32) + b1_ref[...]
    )
    b_ref[...] = jnp.dot(x, wb_ref[...], preferred_element_type=jnp.float32)


def _node_tables(h, W1, b1):
    rows = 1000
    grid = (N_NODES // rows,)
    return pl.pallas_call(
        _tables_body,
        grid=grid,
        in_specs=[
            pl.BlockSpec((rows, H), lambda i: (i, 0)),
            pl.BlockSpec((H, H), lambda i: (0, 0)),
            pl.BlockSpec((H, H), lambda i: (0, 0)),
            pl.BlockSpec((1, H), lambda i: (0, 0)),
        ],
        out_specs=[
            pl.BlockSpec((rows, H), lambda i: (i, 0)),
            pl.BlockSpec((rows, H), lambda i: (i, 0)),
        ],
        out_shape=[
            jax.ShapeDtypeStruct((N_NODES, H), jnp.float32),
            jax.ShapeDtypeStruct((N_NODES, H), jnp.float32),
        ],
    )(h, W1[:H], W1[H:], b1.reshape(1, H))


def _permute(a, perm):
    return lax.gather(
        a, perm[:, None],
        lax.GatherDimensionNumbers(
            offset_dims=(), collapsed_slice_dims=(0,), start_index_map=(0,)
        ),
        slice_sizes=(1,),
        mode=lax.GatherScatterMode.PROMISE_IN_BOUNDS,
        unique_indices=True, indices_are_sorted=False,
    )


@functools.partial(
    pl.kernel,
    out_type=jax.ShapeDtypeStruct((N_EDGES,), jnp.float32),
    mesh=plsc.VectorSubcoreMesh(core_axis_name="c", subcore_axis_name="s"),
    scratch_types=[
        pltpu.VMEM((CH,), jnp.int32),
        pltpu.VMEM((CH,), jnp.int32),
        pltpu.VMEM((CH, H), jnp.float32),
        pltpu.VMEM((H,), jnp.float32),
        pltpu.VMEM((CH,), jnp.float32),
        pltpu.SemaphoreType.DMA,
    ],
)
def _edge_decode(a_hbm, b_hbm, src_hbm, dst_hbm, w2_hbm, out_hbm,
                 src_v, dst_v, z_v, w2_v, out_v, sem):
    wid = lax.axis_index("s") * 2 + lax.axis_index("c")
    base0 = wid * EPW

    pltpu.sync_copy(w2_hbm, w2_v)
    w2r = [w2_v[pl.ds(k * L, L)] for k in range(KV)]
    lane_ids = lax.iota(jnp.int32, L)
    perms = [(lane_ids + sh) & 15 for sh in (8, 4, 2, 1)]
    zero = jnp.zeros((L,), jnp.float32)

    def chunk_body(c, carry):
        base = base0 + c * CH
        pltpu.sync_copy(src_hbm.at[pl.ds(base, CH)], src_v)
        pltpu.sync_copy(dst_hbm.at[pl.ds(base, CH)], dst_v)
        pltpu.async_copy(a_hbm.at[src_v], z_v, sem).wait()
        pltpu.async_copy(b_hbm.at[dst_v], z_v, sem, add=True).wait()

        def group_body(g, gcarry):
            e0 = g * L
            red = zero
            for i in range(L):
                acc = zero
                for k in range(KV):
                    z = z_v[e0 + i, pl.ds(k * L, L)]
                    acc = acc + jnp.maximum(z, 0.0) * w2r[k]
                for p in perms:
                    acc = acc + _permute(acc, p)
                red = jnp.where(lane_ids == i, acc, red)
            out_v[pl.ds(e0, L)] = red
            return gcarry

        lax.fori_loop(0, CH // L, group_body, 0)
        pltpu.sync_copy(out_v, out_hbm.at[pl.ds(base, CH)])
        return carry

    lax.fori_loop(0, NCHUNK, chunk_body, 0)


def kernel(edges, h, W1, b1, W2, b2):
    edges = edges.astype(jnp.int32)
    a_tab, b_tab = _node_tables(h, W1, b1)
    out = _edge_decode(a_tab, b_tab, edges[0], edges[1], W2.reshape(H))
    return out + b2[0]


# speedup vs baseline: 5.7462x; 1.6324x over previous
import functools

import jax
import jax.numpy as jnp
from jax import lax
from jax.experimental import pallas as pl
from jax.experimental.pallas import tpu as pltpu
from jax.experimental.pallas import tpu_sc as plsc

N_NODES = 10000
N_EDGES = 320000
H = 128
L = 16
KV = H // L
NW = 32
EPW = N_EDGES // NW
CH = 80
NCHUNK = EPW // CH


def _tables_body(h_ref, wa_ref, wb_ref, b1_ref, a_ref, b_ref):
    x = h_ref[...]
    a_ref[...] = (
        jnp.dot(x, wa_ref[...], preferred_element_type=jnp.float32) + b1_ref[...]
    )
    b_ref[...] = jnp.dot(x, wb_ref[...], preferred_element_type=jnp.float32)


def _node_tables(h, W1, b1):
    rows = 1000
    grid = (N_NODES // rows,)
    return pl.pallas_call(
        _tables_body,
        grid=grid,
        in_specs=[
            pl.BlockSpec((rows, H), lambda i: (i, 0)),
            pl.BlockSpec((H, H), lambda i: (0, 0)),
            pl.BlockSpec((H, H), lambda i: (0, 0)),
            pl.BlockSpec((1, H), lambda i: (0, 0)),
        ],
        out_specs=[
            pl.BlockSpec((rows, H), lambda i: (i, 0)),
            pl.BlockSpec((rows, H), lambda i: (i, 0)),
        ],
        out_shape=[
            jax.ShapeDtypeStruct((N_NODES, H), jnp.float32),
            jax.ShapeDtypeStruct((N_NODES, H), jnp.float32),
        ],
    )(h, W1[:H], W1[H:], b1.reshape(1, H))


def _permute(a, perm):
    return lax.gather(
        a, perm[:, None],
        lax.GatherDimensionNumbers(
            offset_dims=(), collapsed_slice_dims=(0,), start_index_map=(0,)
        ),
        slice_sizes=(1,),
        mode=lax.GatherScatterMode.PROMISE_IN_BOUNDS,
        unique_indices=True, indices_are_sorted=False,
    )


NBUF = 5


@functools.partial(
    pl.kernel,
    out_type=jax.ShapeDtypeStruct((N_EDGES,), jnp.float32),
    mesh=plsc.VectorSubcoreMesh(core_axis_name="c", subcore_axis_name="s"),
    scratch_types=[
        pltpu.VMEM((NBUF, CH), jnp.int32),
        pltpu.VMEM((NBUF, CH), jnp.int32),
        pltpu.VMEM((NBUF, CH, H), jnp.float32),
        pltpu.VMEM((H,), jnp.float32),
        pltpu.VMEM((NBUF, CH), jnp.float32),
        pltpu.SemaphoreType.DMA,
        pltpu.SemaphoreType.DMA,
        pltpu.SemaphoreType.DMA,
    ],
)
def _edge_decode(a_hbm, b_hbm, src_hbm, dst_hbm, w2_hbm, out_hbm,
                 src_i, dst_i, z3, w2_v, out_b, semA, semB, semO):
    wid = lax.axis_index("s") * 2 + lax.axis_index("c")
    base0 = wid * EPW
    last = NCHUNK - 1

    pltpu.sync_copy(w2_hbm, w2_v)
    w2r = [w2_v[pl.ds(k * L, L)] for k in range(KV)]
    lane_ids = lax.iota(jnp.int32, L)
    perms = [(lane_ids + sh) & 15 for sh in (8, 4, 2, 1)]
    zero = jnp.zeros((L,), jnp.float32)

    def load_idx(c, slot):
        pltpu.sync_copy(src_hbm.at[pl.ds(base0 + c * CH, CH)], src_i.at[slot])
        pltpu.sync_copy(dst_hbm.at[pl.ds(base0 + c * CH, CH)], dst_i.at[slot])

    def start_ga(slot):
        pltpu.async_copy(a_hbm.at[src_i.at[slot]], z3.at[slot], semA)

    def wait_ga(slot):
        pltpu.make_async_copy(a_hbm.at[pl.ds(0, CH)], z3.at[slot], semA).wait()

    def start_gb(slot):
        pltpu.async_copy(b_hbm.at[dst_i.at[slot]], z3.at[slot], semB, add=True)

    def wait_gb(slot):
        pltpu.make_async_copy(b_hbm.at[pl.ds(0, CH)], z3.at[slot], semB).wait()

    def wait_out(slot):
        pltpu.make_async_copy(out_b.at[slot], out_hbm.at[pl.ds(0, CH)], semO).wait()

    load_idx(0, 0)
    start_ga(0)
    load_idx(1, 1)
    start_ga(1)
    wait_ga(0)
    start_gb(0)

    def blk_body(blk, carry):
        for j in range(NBUF):
            c = blk * NBUF + j
            s1 = (j + 1) % NBUF
            s2 = (j + 2) % NBUF
            wait_ga(s1)
            start_gb(s1)
            c2 = jnp.minimum(c + 2, last)
            load_idx(c2, s2)
            start_ga(s2)
            wait_gb(j)

            @pl.when(c >= NBUF)
            def _():
                wait_out(j)

            def group_body(g, gcarry):
                e0 = g * L
                red = zero
                for i in range(L):
                    acc = zero
                    for k in range(KV):
                        z = z3[j, e0 + i, pl.ds(k * L, L)]
                        acc = acc + jnp.maximum(z, 0.0) * w2r[k]
                    for p in perms:
                        acc = acc + _permute(acc, p)
                    red = jnp.where(lane_ids == i, acc, red)
                out_b[j, pl.ds(e0, L)] = red
                return gcarry

            lax.fori_loop(0, CH // L, group_body, 0)
            pltpu.async_copy(out_b.at[j], out_hbm.at[pl.ds(base0 + c * CH, CH)], semO)
        return carry

    lax.fori_loop(0, NCHUNK // NBUF, blk_body, 0)

    wait_ga(0)
    wait_gb(1)
    for j in range(NBUF):
        wait_out(j)


def kernel(edges, h, W1, b1, W2, b2):
    edges = edges.astype(jnp.int32)
    a_tab, b_tab = _node_tables(h, W1, b1)
    out = _edge_decode(a_tab, b_tab, edges[0], edges[1], W2.reshape(H))
    return out + b2[0]
